# baseline (speedup 1.0000x reference)
import jax
import jax.numpy as jnp
from jax import lax
from jax.experimental import pallas as pl
from jax.experimental.pallas import tpu as pltpu
from jax.experimental.pallas import tpu_sc as plsc

B = 16384
D = 64
L = 16
NC = 2
NS = 16
NW = NC * NS
ROWS = B // NW
CROWS = B // NC
HROWS = CROWS // 2
CH = 128
NCH = ROWS // CH
NG = ROWS // L
WB = D + L


def _sc_body(x_hbm, z_hbm, idx_hbm, wb_hbm, emb1_hbm, emb2_hbm,
             out_hbm, idx_v, a_v, b_v, x_v, z_v, wb_v, out_v,
             sh, sem, sem_sh):
    cid = lax.axis_index("c")
    sid = lax.axis_index("s")
    wid = cid * NS + sid
    base = wid * ROWS
    cbase = cid * CROWS

    copies = []

    def fire(src, dst):
        c = pltpu.make_async_copy(src, dst, sem)
        c.start()
        copies.append(c)

    pltpu.sync_copy(idx_hbm.at[pl.ds(wid * NCH, NCH)], idx_v)
    for j in range(NCH):
        fire(emb1_hbm.at[idx_v.at[j]], a_v.at[pl.ds(j * CH, CH)])
        fire(emb2_hbm.at[idx_v.at[j]], b_v.at[pl.ds(j * CH, CH)])
    fire(wb_hbm, wb_v)

    for arr_hbm, dst_v in ((z_hbm, z_v), (x_hbm, x_v)):
        for h in range(2):
            @pl.when(sid == 0)
            def _dma(arr_hbm=arr_hbm, h=h):
                c = pltpu.make_async_copy(
                    arr_hbm.at[pl.ds(cbase + h * HROWS, HROWS)], sh, sem_sh)
                c.start()
                c.wait()

            plsc.subcore_barrier()

            @pl.when(sid // 8 == h)
            def _pull(dst_v=dst_v, h=h):
                pltpu.sync_copy(
                    sh.at[pl.ds((sid % 8) * ROWS, ROWS)], dst_v)

            plsc.subcore_barrier()

    for c in copies:
        c.wait()

    wregs = [wb_v[0, pl.ds(k * L, L)] for k in range(D // L)]
    bias_vec = wb_v[0, pl.ds(D, L)]
    lanes = lax.iota(jnp.int32, L)

    def group(g, carry):
        out16 = jnp.zeros((L,), jnp.float32)
        for rr in range(L):
            r = g * L + rr
            acc = z_v[r, pl.ds(0, L)] * a_v[r, pl.ds(0, L)]
            for k in range(1, D // L):
                acc = acc + z_v[r, pl.ds(k * L, L)] * a_v[r, pl.ds(k * L, L)]
            for k in range(D // L):
                acc = acc + x_v[r, pl.ds(k * L, L)] * wregs[k]
            s = jnp.sum(acc)
            out16 = jnp.where(lanes == rr, s, out16)
        bv = plsc.load_gather(b_v, [g * L + lanes, jnp.zeros((L,), jnp.int32)])
        out_v[pl.dslice(g * L, L)] = out16 + bv + bias_vec
        return carry

    lax.fori_loop(0, NG, group, 0)

    pltpu.sync_copy(out_v, out_hbm.at[pl.ds(base, ROWS)])


def _build_sc():
    mesh = plsc.VectorSubcoreMesh(core_axis_name="c", subcore_axis_name="s")
    return pl.kernel(
        _sc_body,
        out_type=jax.ShapeDtypeStruct((B,), jnp.float32),
        mesh=mesh,
        compiler_params=pltpu.CompilerParams(
            needs_layout_passes=False, use_tc_tiling_on_sc=False),
        scratch_types=[
            pltpu.VMEM((NCH, CH), jnp.int32),
            pltpu.VMEM((ROWS, D), jnp.float32),
            pltpu.VMEM((ROWS, 1), jnp.float32),
            pltpu.VMEM((ROWS, D), jnp.float32),
            pltpu.VMEM((ROWS, D), jnp.float32),
            pltpu.VMEM((1, WB), jnp.float32),
            pltpu.VMEM((ROWS,), jnp.float32),
            pltpu.VMEM_SHARED((HROWS, D), jnp.float32),
            pltpu.SemaphoreType.DMA,
            pltpu.SemaphoreType.DMA,
        ],
    )


_sc_kernel = _build_sc()


@jax.jit
def kernel(x, z, idx, W_f, b_f, emb1, emb2):
    wb = jnp.concatenate([W_f, jnp.broadcast_to(b_f, (1, L))], axis=1)
    idx2 = idx.astype(jnp.int32).reshape(NW * NCH, CH)
    out = _sc_kernel(x, z, idx2, wb, emb1, emb2)
    return out.reshape(B, 1)


# speedup vs baseline: 1.6013x; 1.6013x over previous
import jax
import jax.numpy as jnp
from jax import lax
from jax.experimental import pallas as pl
from jax.experimental.pallas import tpu as pltpu
from jax.experimental.pallas import tpu_sc as plsc

B = 16384
D = 64
L = 16
NC = 2
NS = 16
NW = NC * NS
ROWS = B // NW
CROWS = B // NC
HROWS = CROWS // 2
CH = 128
NCH = ROWS // CH
NG = ROWS // L
WB = D + L


def _sc_body(x_hbm, z_hbm, idx_hbm, wb_hbm, emb1_hbm,
             out_hbm, idx_v, a_v, x_v, z_v, wb_v, out_v,
             sh, sem, sem_sh):
    cid = lax.axis_index("c")
    sid = lax.axis_index("s")
    wid = cid * NS + sid
    base = wid * ROWS
    cbase = cid * CROWS

    copies = []

    def fire(src, dst):
        c = pltpu.make_async_copy(src, dst, sem)
        c.start()
        copies.append(c)

    pltpu.sync_copy(idx_hbm.at[pl.ds(wid * NCH, NCH)], idx_v)
    for j in range(NCH):
        fire(emb1_hbm.at[idx_v.at[j]], a_v.at[pl.ds(j * CH, CH)])
    fire(wb_hbm, wb_v)

    for arr_hbm, dst_v in ((z_hbm, z_v), (x_hbm, x_v)):
        for h in range(2):
            @pl.when(sid == 0)
            def _dma(arr_hbm=arr_hbm, h=h):
                c = pltpu.make_async_copy(
                    arr_hbm.at[pl.ds(cbase + h * HROWS, HROWS)], sh, sem_sh)
                c.start()
                c.wait()

            plsc.subcore_barrier()

            @pl.when(sid // 8 == h)
            def _pull(dst_v=dst_v, h=h):
                pltpu.sync_copy(
                    sh.at[pl.ds((sid % 8) * ROWS, ROWS)], dst_v)

            plsc.subcore_barrier()

    for c in copies:
        c.wait()

    wregs = [wb_v[0, pl.ds(k * L, L)] for k in range(D // L)]
    bias_vec = wb_v[0, pl.ds(D, L)]
    lanes = lax.iota(jnp.int32, L)

    def group(g, carry):
        out16 = jnp.zeros((L,), jnp.float32)
        for rr in range(L):
            r = g * L + rr
            acc = z_v[r, pl.ds(0, L)] * a_v[r, pl.ds(0, L)]
            for k in range(1, D // L):
                acc = acc + z_v[r, pl.ds(k * L, L)] * a_v[r, pl.ds(k * L, L)]
            for k in range(D // L):
                acc = acc + x_v[r, pl.ds(k * L, L)] * wregs[k]
            s = jnp.sum(acc)
            out16 = jnp.where(lanes == rr, s, out16)
        out_v[pl.dslice(g * L, L)] = out16 + bias_vec
        return carry

    lax.fori_loop(0, NG, group, 0)

    pltpu.sync_copy(out_v, out_hbm.at[pl.ds(base, ROWS)])


def _build_sc():
    mesh = plsc.VectorSubcoreMesh(core_axis_name="c", subcore_axis_name="s")
    return pl.kernel(
        _sc_body,
        out_type=jax.ShapeDtypeStruct((B,), jnp.float32),
        mesh=mesh,
        compiler_params=pltpu.CompilerParams(
            needs_layout_passes=False, use_tc_tiling_on_sc=False),
        scratch_types=[
            pltpu.VMEM((NCH, CH), jnp.int32),
            pltpu.VMEM((ROWS, D), jnp.float32),
            pltpu.VMEM((ROWS, D), jnp.float32),
            pltpu.VMEM((ROWS, D), jnp.float32),
            pltpu.VMEM((1, WB), jnp.float32),
            pltpu.VMEM((ROWS,), jnp.float32),
            pltpu.VMEM_SHARED((HROWS, D), jnp.float32),
            pltpu.SemaphoreType.DMA,
            pltpu.SemaphoreType.DMA,
        ],
    )


_sc_kernel = _build_sc()


@jax.jit
def kernel(x, z, idx, W_f, b_f, emb1, emb2):
    wb = jnp.concatenate([W_f, jnp.broadcast_to(b_f, (1, L))], axis=1)
    idx2 = idx.astype(jnp.int32).reshape(NW * NCH, CH)
    rand = _sc_kernel(x, z, idx2, wb, emb1)
    return rand.reshape(B, 1) + jnp.take(emb2, idx, axis=0)


# speedup vs baseline: 1.6134x; 1.0075x over previous
import jax
import jax.numpy as jnp
from jax import lax
from jax.experimental import pallas as pl
from jax.experimental.pallas import tpu as pltpu
from jax.experimental.pallas import tpu_sc as plsc

B = 16384
D = 64
L = 16
NC = 2
NS = 16
NW = NC * NS
ROWS = B // NW
CROWS = B // NC
HROWS = CROWS // 2
QROWS = CROWS // 4
CH = 128
NCH = ROWS // CH
NG = ROWS // L
WB = D + L


def _sc_body(x_hbm, z_hbm, idx_hbm, wb_hbm, emb1_hbm,
             out_hbm, idx_v, a_v, x_v, z_v, wb_v, out_v,
             sh_a, sh_b, sem, sem_sh):
    cid = lax.axis_index("c")
    sid = lax.axis_index("s")
    wid = cid * NS + sid
    base = wid * ROWS
    cbase = cid * CROWS

    copies = []

    def fire(src, dst):
        c = pltpu.make_async_copy(src, dst, sem)
        c.start()
        copies.append(c)

    pltpu.sync_copy(idx_hbm.at[pl.ds(wid * NCH, NCH)], idx_v)
    for j in range(NCH):
        fire(emb1_hbm.at[idx_v.at[j]], a_v.at[pl.ds(j * CH, CH)])
    fire(wb_hbm, wb_v)

    bufs = (sh_a, sh_b)
    phases = [(z_hbm, z_v, h) for h in range(4)] + \
             [(x_hbm, x_v, h) for h in range(4)]

    def dma_of(p):
        arr_hbm, _, h = phases[p]
        return pltpu.make_async_copy(
            arr_hbm.at[pl.ds((cbase + h * QROWS) * D, QROWS * D)],
            bufs[p % 2], sem_sh)

    @pl.when(sid == 0)
    def _dma0():
        dma_of(0).start()

    for p in range(len(phases)):
        _, dst_v, h = phases[p]

        @pl.when(sid == 0)
        def _wait_p(p=p):
            dma_of(p).wait()

        plsc.subcore_barrier()

        if p + 1 < len(phases):
            @pl.when(sid == 0)
            def _next_p(p=p):
                dma_of(p + 1).start()

        @pl.when(sid // 4 == h)
        def _pull(dst_v=dst_v, p=p, h=h):
            pltpu.sync_copy(
                bufs[p % 2].at[pl.ds((sid % 4) * ROWS * D, ROWS * D)], dst_v)

        plsc.subcore_barrier()

    for c in copies:
        c.wait()

    wregs = [wb_v[0, pl.ds(k * L, L)] for k in range(D // L)]
    bias_vec = wb_v[0, pl.ds(D, L)]
    lanes = lax.iota(jnp.int32, L)

    def group(g, carry):
        out16 = jnp.zeros((L,), jnp.float32)
        for rr in range(L):
            r = g * L + rr
            acc = z_v[pl.ds(r * D, L)] * a_v[r, pl.ds(0, L)]
            for k in range(1, D // L):
                acc = acc + z_v[pl.ds(r * D + k * L, L)] * a_v[r, pl.ds(k * L, L)]
            for k in range(D // L):
                acc = acc + x_v[pl.ds(r * D + k * L, L)] * wregs[k]
            s = jnp.sum(acc)
            out16 = jnp.where(lanes == rr, s, out16)
        out_v[pl.dslice(g * L, L)] = out16 + bias_vec
        return carry

    lax.fori_loop(0, NG, group, 0)

    pltpu.sync_copy(out_v, out_hbm.at[pl.ds(base, ROWS)])


def _build_sc():
    mesh = plsc.VectorSubcoreMesh(core_axis_name="c", subcore_axis_name="s")
    return pl.kernel(
        _sc_body,
        out_type=jax.ShapeDtypeStruct((B,), jnp.float32),
        mesh=mesh,
        compiler_params=pltpu.CompilerParams(
            needs_layout_passes=False, use_tc_tiling_on_sc=False),
        scratch_types=[
            pltpu.VMEM((NCH, CH), jnp.int32),
            pltpu.VMEM((ROWS, D), jnp.float32),
            pltpu.VMEM((ROWS * D,), jnp.float32),
            pltpu.VMEM((ROWS * D,), jnp.float32),
            pltpu.VMEM((1, WB), jnp.float32),
            pltpu.VMEM((ROWS,), jnp.float32),
            pltpu.VMEM_SHARED((QROWS * D,), jnp.float32),
            pltpu.VMEM_SHARED((QROWS * D,), jnp.float32),
            pltpu.SemaphoreType.DMA,
            pltpu.SemaphoreType.DMA,
        ],
    )


_sc_kernel = _build_sc()


@jax.jit
def kernel(x, z, idx, W_f, b_f, emb1, emb2):
    wb = jnp.concatenate([W_f, jnp.broadcast_to(b_f, (1, L))], axis=1)
    idx2 = idx.astype(jnp.int32).reshape(NW * NCH, CH)
    rand = _sc_kernel(x.reshape(-1), z.reshape(-1), idx2, wb, emb1)
    return rand.reshape(B, 1) + jnp.take(emb2, idx, axis=0)


# speedup vs baseline: 1.7215x; 1.0670x over previous
import jax
import jax.numpy as jnp
from jax import lax
from jax.experimental import pallas as pl
from jax.experimental.pallas import tpu as pltpu
from jax.experimental.pallas import tpu_sc as plsc

B = 16384
D = 64
L = 16
NC = 2
NS = 16
NW = NC * NS
ROWS = B // NW
CROWS = B // NC
HROWS = CROWS // 2
QROWS = CROWS // 4
CH = 128
NCH = ROWS // CH
NG = ROWS // L
WB = D + L


def _sc_body(x_hbm, z_hbm, idx_hbm, wb_hbm, emb1_hbm, emb2_hbm,
             out_hbm, idx_v, a_v, b_v, x_v, z_v, wb_v, out_v,
             sh_a, sh_b, sem, sem_sh):
    cid = lax.axis_index("c")
    sid = lax.axis_index("s")
    wid = cid * NS + sid
    base = wid * ROWS
    cbase = cid * CROWS

    copies = []

    def fire(src, dst):
        c = pltpu.make_async_copy(src, dst, sem)
        c.start()
        copies.append(c)

    pltpu.sync_copy(idx_hbm.at[pl.ds(wid * NCH, NCH)], idx_v)
    for j in range(NCH):
        fire(emb1_hbm.at[idx_v.at[j]], a_v.at[pl.ds(j * CH, CH)])
        fire(emb2_hbm.at[idx_v.at[j]], b_v.at[pl.ds(j * CH, CH)])
    fire(wb_hbm, wb_v)

    bufs = (sh_a, sh_b)
    phases = [(z_hbm, z_v, h) for h in range(4)] + \
             [(x_hbm, x_v, h) for h in range(4)]

    def dma_of(p):
        arr_hbm, _, h = phases[p]
        return pltpu.make_async_copy(
            arr_hbm.at[pl.ds((cbase + h * QROWS) * D, QROWS * D)],
            bufs[p % 2], sem_sh)

    @pl.when(sid == 0)
    def _dma0():
        dma_of(0).start()

    for p in range(len(phases)):
        _, dst_v, h = phases[p]

        @pl.when(sid == 0)
        def _wait_p(p=p):
            dma_of(p).wait()

        plsc.subcore_barrier()

        if p + 1 < len(phases):
            @pl.when(sid == 0)
            def _next_p(p=p):
                dma_of(p + 1).start()

        @pl.when(sid // 4 == h)
        def _pull(dst_v=dst_v, p=p, h=h):
            pltpu.sync_copy(
                bufs[p % 2].at[pl.ds((sid % 4) * ROWS * D, ROWS * D)], dst_v)

        plsc.subcore_barrier()

    for c in copies:
        c.wait()

    wregs = [wb_v[0, pl.ds(k * L, L)] for k in range(D // L)]
    bias_vec = wb_v[0, pl.ds(D, L)]
    lanes = lax.iota(jnp.int32, L)

    def group(g, carry):
        out16 = jnp.zeros((L,), jnp.float32)
        for rr in range(L):
            r = g * L + rr
            acc = z_v[pl.ds(r * D, L)] * a_v[r, pl.ds(0, L)]
            for k in range(1, D // L):
                acc = acc + z_v[pl.ds(r * D + k * L, L)] * a_v[r, pl.ds(k * L, L)]
            for k in range(D // L):
                acc = acc + x_v[pl.ds(r * D + k * L, L)] * wregs[k]
            s = jnp.sum(acc)
            out16 = jnp.where(lanes == rr, s, out16)
        bv = b_v[pl.dslice(g * L, L)]
        out_v[pl.dslice(g * L, L)] = out16 + bv + bias_vec
        return carry

    lax.fori_loop(0, NG, group, 0)

    pltpu.sync_copy(out_v, out_hbm.at[pl.ds(base, ROWS)])


def _build_sc():
    mesh = plsc.VectorSubcoreMesh(core_axis_name="c", subcore_axis_name="s")
    return pl.kernel(
        _sc_body,
        out_type=jax.ShapeDtypeStruct((B,), jnp.float32),
        mesh=mesh,
        compiler_params=pltpu.CompilerParams(
            needs_layout_passes=False, use_tc_tiling_on_sc=False),
        scratch_types=[
            pltpu.VMEM((NCH, CH), jnp.int32),
            pltpu.VMEM((ROWS, D), jnp.float32),
            pltpu.VMEM((ROWS,), jnp.float32),
            pltpu.VMEM((ROWS * D,), jnp.float32),
            pltpu.VMEM((ROWS * D,), jnp.float32),
            pltpu.VMEM((1, WB), jnp.float32),
            pltpu.VMEM((ROWS,), jnp.float32),
            pltpu.VMEM_SHARED((QROWS * D,), jnp.float32),
            pltpu.VMEM_SHARED((QROWS * D,), jnp.float32),
            pltpu.SemaphoreType.DMA,
            pltpu.SemaphoreType.DMA,
        ],
    )


_sc_kernel = _build_sc()


@jax.jit
def kernel(x, z, idx, W_f, b_f, emb1, emb2):
    wb = jnp.concatenate([W_f, jnp.broadcast_to(b_f, (1, L))], axis=1)
    idx2 = idx.astype(jnp.int32).reshape(NW * NCH, CH)
    out = _sc_kernel(x.reshape(-1), z.reshape(-1), idx2, wb, emb1,
                     emb2.reshape(-1))
    return out.reshape(B, 1)


# speedup vs baseline: 1.7248x; 1.0019x over previous
import jax
import jax.numpy as jnp
from jax import lax
from jax.experimental import pallas as pl
from jax.experimental.pallas import tpu as pltpu
from jax.experimental.pallas import tpu_sc as plsc

B = 16384
D = 64
L = 16
NC = 2
NS = 16
NW = NC * NS
ROWS = B // NW
CROWS = B // NC
QROWS = CROWS // 4
CH = 128
NCH = ROWS // CH
NG = ROWS // L
WB = D + L


def _sc_body(x_hbm, z_hbm, idx_hbm, wb_hbm, emb1_hbm, emb2_hbm,
             out_hbm, idx_v, a_v, b_v, x_v, z_v, wb_v, out_v,
             sh_a, sh_b, sem, sem_sh):
    cid = lax.axis_index("c")
    sid = lax.axis_index("s")
    wid = cid * NS + sid
    base = wid * ROWS
    cbase = cid * CROWS

    copies = []

    def fire(src, dst):
        c = pltpu.make_async_copy(src, dst, sem)
        c.start()
        copies.append(c)

    pltpu.sync_copy(idx_hbm.at[pl.ds(wid * NCH, NCH)], idx_v)
    for j in range(NCH):
        fire(emb1_hbm.at[idx_v.at[j]], a_v.at[pl.ds(j * CH, CH)])
        fire(emb2_hbm.at[idx_v.at[j]], b_v.at[pl.ds(j * CH, CH)])
    fire(wb_hbm, wb_v)

    bufs = (sh_a, sh_b)
    phases = [(z_hbm, z_v, h) for h in range(4)] + \
             [(x_hbm, x_v, h) for h in range(4)]

    def dma_of(p):
        arr_hbm, _, h = phases[p]
        return pltpu.make_async_copy(
            arr_hbm.at[pl.ds((cbase + h * QROWS) * D, QROWS * D)],
            bufs[p % 2], sem_sh)

    @pl.when(sid == 0)
    def _dma0():
        dma_of(0).start()

    for p in range(len(phases)):
        _, dst_v, h = phases[p]

        @pl.when(sid == 0)
        def _wait_p(p=p):
            dma_of(p).wait()

        plsc.subcore_barrier()

        if p + 1 < len(phases):
            @pl.when(sid == 0)
            def _next_p(p=p):
                dma_of(p + 1).start()

        @pl.when(sid // 4 == h)
        def _pull(dst_v=dst_v, p=p, h=h):
            pltpu.sync_copy(
                bufs[p % 2].at[pl.ds((sid % 4) * ROWS * D, ROWS * D)], dst_v)

        plsc.subcore_barrier()

    for c in copies:
        c.wait()

    wregs = [wb_v[0, pl.ds(k * L, L)] for k in range(D // L)]
    bias_vec = wb_v[0, pl.ds(D, L)]
    lanes = lax.iota(jnp.int32, L)

    def group(g, carry):
        out16 = jnp.zeros((L,), jnp.float32)
        for rr in range(L):
            r = g * L + rr
            acc = z_v[pl.ds(r * D, L)] * a_v[r, pl.ds(0, L)]
            for k in range(1, D // L):
                acc = acc + z_v[pl.ds(r * D + k * L, L)] * a_v[r, pl.ds(k * L, L)]
            for k in range(D // L):
                acc = acc + x_v[pl.ds(r * D + k * L, L)] * wregs[k]
            s = jnp.sum(acc)
            out16 = jnp.where(lanes == rr, s, out16)
        bv = b_v[pl.dslice(g * L, L)]
        out_v[pl.dslice(g * L, L)] = out16 + bv + bias_vec
        return carry

    lax.fori_loop(0, NG, group, 0)

    pltpu.sync_copy(out_v, out_hbm.at[pl.ds(base, ROWS)])


def _build_sc():
    mesh = plsc.VectorSubcoreMesh(core_axis_name="c", subcore_axis_name="s")
    return pl.kernel(
        _sc_body,
        out_type=jax.ShapeDtypeStruct((B,), jnp.float32),
        mesh=mesh,
        compiler_params=pltpu.CompilerParams(
            needs_layout_passes=False, use_tc_tiling_on_sc=False),
        scratch_types=[
            pltpu.VMEM((NCH, CH), jnp.int32),
            pltpu.VMEM((ROWS, D), jnp.float32),
            pltpu.VMEM((ROWS,), jnp.float32),
            pltpu.VMEM((ROWS * D,), jnp.float32),
            pltpu.VMEM((ROWS * D,), jnp.float32),
            pltpu.VMEM((1, WB), jnp.float32),
            pltpu.VMEM((ROWS,), jnp.float32),
            pltpu.VMEM_SHARED((QROWS * D,), jnp.float32),
            pltpu.VMEM_SHARED((QROWS * D,), jnp.float32),
            pltpu.SemaphoreType.DMA,
            pltpu.SemaphoreType.DMA,
        ],
    )


_sc_kernel = _build_sc()


@jax.jit
def kernel(x, z, idx, W_f, b_f, emb1, emb2):
    wb = jnp.concatenate([W_f, jnp.broadcast_to(b_f, (1, L))], axis=1)
    idx2 = idx.astype(jnp.int32).reshape(NW * NCH, CH)
    out = _sc_kernel(x.reshape(-1), z.reshape(-1), idx2, wb, emb1,
                     emb2.reshape(-1))
    return out.reshape(B, 1)
